# Initial kernel scaffold; baseline (speedup 1.0000x reference)
import jax
import jax.numpy as jnp
from jax.experimental import pallas as pl


def kernel(token_ids, weight):
    raise NotImplementedError("write your pallas kernel here")


# speedup vs baseline: 4.9483x; 4.9483x over previous
import functools

import jax
import jax.numpy as jnp
from jax import lax
from jax.experimental import pallas as pl
from jax.experimental.pallas import tpu as pltpu
from jax.experimental.pallas import tpu_sc as plsc

_LANE = 128
_K = 16


@functools.lru_cache(maxsize=None)
def _make_gather(nb, d):
    info = plsc.get_sparse_core_info()
    nc, ns = info.num_cores, info.num_subcores
    nw = nc * ns
    blocks_per_w = nb // nw
    steps = blocks_per_w // _K
    assert blocks_per_w * nw == nb and steps * _K == blocks_per_w
    mesh = plsc.VectorSubcoreMesh(core_axis_name="c", subcore_axis_name="s")

    @functools.partial(
        pl.kernel,
        mesh=mesh,
        out_type=jax.ShapeDtypeStruct((nb, _LANE, d), jnp.float32),
        scratch_types=[
            pltpu.VMEM((_K, _LANE), jnp.int32),
            pltpu.VMEM((_K, _LANE, d), jnp.float32),
            pltpu.SemaphoreType.DMA,
        ],
        compiler_params=pltpu.CompilerParams(use_tc_tiling_on_sc=False),
    )
    def gather_kernel(idx_hbm, table_hbm, out_hbm, idx_v, rows_v, sem):
        wid = lax.axis_index("s") * nc + lax.axis_index("c")
        base = wid * blocks_per_w

        def body(g, carry):
            blk = base + g * _K
            pltpu.sync_copy(idx_hbm.at[pl.ds(blk, _K)], idx_v)
            descs = [
                pltpu.async_copy(table_hbm.at[idx_v.at[j]], rows_v.at[j], sem)
                for j in range(_K)
            ]
            for dsc in descs:
                dsc.wait()
            pltpu.sync_copy(rows_v, out_hbm.at[pl.ds(blk, _K)])
            return carry

        lax.fori_loop(0, steps, body, 0)

    return gather_kernel


def kernel(token_ids, weight):
    b, s = token_ids.shape
    d = weight.shape[1]
    flat = token_ids.reshape(-1).astype(jnp.int32)
    nb = flat.shape[0] // _LANE
    blocks = flat.reshape(nb, _LANE)
    out = _make_gather(nb, d)(blocks, weight)
    return out.reshape(b, s, d)


# speedup vs baseline: 4.9489x; 1.0001x over previous
import functools

import jax
import jax.numpy as jnp
from jax import lax
from jax.experimental import pallas as pl
from jax.experimental.pallas import tpu as pltpu
from jax.experimental.pallas import tpu_sc as plsc

_LANE = 128
_K = 8
_NS = 2


@functools.lru_cache(maxsize=None)
def _make_gather(nb, d):
    info = plsc.get_sparse_core_info()
    nc, ns = info.num_cores, info.num_subcores
    nw = nc * ns
    blocks_per_w = nb // nw
    chunks = blocks_per_w // _K
    T = chunks // _NS
    assert blocks_per_w * nw == nb and chunks * _K == blocks_per_w
    assert T * _NS == chunks
    mesh = plsc.VectorSubcoreMesh(core_axis_name="c", subcore_axis_name="s")

    @functools.partial(
        pl.kernel,
        mesh=mesh,
        out_type=jax.ShapeDtypeStruct((nb, _LANE, d), jnp.float32),
        scratch_types=[
            pltpu.VMEM((_NS, _K, _LANE), jnp.int32),
            pltpu.VMEM((_NS, _K, _LANE, d), jnp.float32),
            pltpu.SemaphoreType.DMA((_NS,)),
            pltpu.SemaphoreType.DMA((_NS,)),
        ],
        compiler_params=pltpu.CompilerParams(use_tc_tiling_on_sc=False),
    )
    def gather_kernel(idx_hbm, table_hbm, out_hbm, idx_v, rows_v, gsem, ssem):
        wid = lax.axis_index("s") * nc + lax.axis_index("c")
        base = wid * blocks_per_w

        def fire_chunk(c, b):
            blk = base + c * _K
            pltpu.sync_copy(idx_hbm.at[pl.ds(blk, _K)], idx_v.at[b])
            for j in range(_K):
                pltpu.async_copy(
                    table_hbm.at[idx_v.at[b].at[j]], rows_v.at[b].at[j],
                    gsem.at[b])

        def wait_gathers(b):
            for j in range(_K):
                pltpu.make_async_copy(
                    table_hbm.at[idx_v.at[b].at[j]], rows_v.at[b].at[j],
                    gsem.at[b]).wait()

        def fire_store(c, b):
            blk = base + c * _K
            pltpu.async_copy(rows_v.at[b], out_hbm.at[pl.ds(blk, _K)],
                             ssem.at[b])

        def wait_store(b):
            pltpu.make_async_copy(rows_v.at[b], out_hbm.at[pl.ds(0, _K)],
                                  ssem.at[b]).wait()

        def body(t, carry):
            for b in range(_NS):
                c = t * _NS + b

                @pl.when(t > 0)
                def _():
                    wait_store(b)

                fire_chunk(c, b)
                prev_b = 1 - b
                if b == 0:
                    @pl.when(t > 0)
                    def _():
                        wait_gathers(prev_b)
                        fire_store(t * _NS - 1, prev_b)
                else:
                    wait_gathers(prev_b)
                    fire_store(t * _NS, prev_b)
            return carry

        lax.fori_loop(0, T, body, 0)
        wait_gathers(1)
        fire_store(chunks - 1, 1)
        wait_store(0)
        wait_store(1)

    return gather_kernel


def kernel(token_ids, weight):
    b, s = token_ids.shape
    d = weight.shape[1]
    flat = token_ids.reshape(-1).astype(jnp.int32)
    nb = flat.shape[0] // _LANE
    blocks = flat.reshape(nb, _LANE)
    out = _make_gather(nb, d)(blocks, weight)
    return out.reshape(b, s, d)
